# Initial kernel scaffold; baseline (speedup 1.0000x reference)
import jax
import jax.numpy as jnp
from jax.experimental import pallas as pl


def kernel(values_tree, indices, values):
    raise NotImplementedError("write your pallas kernel here")


# speedup vs baseline: 30.7176x; 30.7176x over previous
import jax
import jax.numpy as jnp
from jax import lax
from jax.experimental import pallas as pl
from jax.experimental.pallas import tpu as pltpu
from jax.experimental.pallas import tpu_sc as plsc

CAPACITY = 1048576
LEFTMOST = 1048576
TREE = 2 * CAPACITY
DEPTH = 19
BATCH = 16384
NTILES = 16
PER_TILE = BATCH // NTILES
ROWS = PER_TILE // 128
SUB = 128 // 16
LEAF_CHUNK = CAPACITY // NTILES
INT_CHUNK = CAPACITY // NTILES


def _body(tree_hbm, sidx_hbm, wval_hbm, sibv_hbm, out_hbm,
          sp, idx_v, val_v, sib_v, pidx_v, lidx_v, ridx_v,
          left_v, right_v, sum_v, fix_v, ifix_v, sem, sem_cp):
    s = lax.axis_index("s").astype(jnp.int32)
    J = [jnp.int32(j) for j in range(ROWS)]

    cp = pltpu.async_copy(
        tree_hbm.at[pl.ds(CAPACITY + s * LEAF_CHUNK, LEAF_CHUNK)],
        out_hbm.at[pl.ds(CAPACITY + s * LEAF_CHUNK, LEAF_CHUNK)], sem_cp)
    st = pltpu.async_copy(
        tree_hbm.at[pl.ds(s * INT_CHUNK, INT_CHUNK)],
        sp.at[pl.ds(s * INT_CHUNK, INT_CHUNK)], sem_cp)
    pltpu.sync_copy(sidx_hbm.at[s], idx_v)
    pltpu.sync_copy(wval_hbm.at[s], val_v)
    pltpu.sync_copy(sibv_hbm.at[s], sib_v)
    cp.wait()
    st.wait()
    plsc.subcore_barrier()

    leaf = [pltpu.async_copy(val_v.at[jj], out_hbm.at[idx_v.at[jj]], sem_cp)
            for jj in J]

    one = jnp.ones((16,), jnp.int32)
    for jj in J:
        for i in range(SUB):
            sl = pl.ds(i * 16, 16)
            v = idx_v[jj, sl]
            lidx_v[jj, sl] = lax.bitwise_and(v, ~one)
            ridx_v[jj, sl] = lax.bitwise_or(v, one)
            pidx_v[jj, sl] = lax.shift_right_logical(v, one)
    g1 = []
    for jj in J:
        g1.append(pltpu.async_copy(tree_hbm.at[lidx_v.at[jj]], left_v.at[jj], sem))
        g1.append(pltpu.async_copy(tree_hbm.at[ridx_v.at[jj]], right_v.at[jj], sem))
    for h in g1:
        h.wait()
    for h in leaf:
        h.wait()
    for jj in J:
        for i in range(SUB):
            sl = pl.ds(i * 16, 16)
            v = idx_v[jj, sl]
            odd = lax.bitwise_and(v, one) == one
            wv = val_v[jj, sl]
            sb = sib_v[jj, sl]
            lv = jnp.where(odd, sb, wv)
            rv = jnp.where(odd, wv, sb)
            lv = jnp.where(jnp.isnan(lv), left_v[jj, sl], lv)
            rv = jnp.where(jnp.isnan(rv), right_v[jj, sl], rv)
            sum_v[jj, sl] = lv + rv
    s1 = [pltpu.async_copy(sum_v.at[jj], sp.at[pidx_v.at[jj]], sem)
          for jj in J]
    for h in s1:
        h.wait()
    plsc.subcore_barrier()

    def level(k, carry):
        for jj in J:
            for i in range(SUB):
                sl = pl.ds(i * 16, 16)
                v = pidx_v[jj, sl]
                lidx_v[jj, sl] = lax.bitwise_and(v, ~one)
                ridx_v[jj, sl] = lax.bitwise_or(v, one)
                pidx_v[jj, sl] = lax.shift_right_logical(v, one)
        gathers = []
        for jj in J:
            gathers.append(
                pltpu.async_copy(sp.at[lidx_v.at[jj]], left_v.at[jj], sem))
            gathers.append(
                pltpu.async_copy(sp.at[ridx_v.at[jj]], right_v.at[jj], sem))
        for h in gathers:
            h.wait()
        for jj in J:
            for i in range(SUB):
                sl = pl.ds(i * 16, 16)
                sum_v[jj, sl] = left_v[jj, sl] + right_v[jj, sl]
        scatters = [pltpu.async_copy(sum_v.at[jj], sp.at[pidx_v.at[jj]], sem)
                    for jj in J]
        for h in scatters:
            h.wait()
        plsc.subcore_barrier()
        return carry

    lax.fori_loop(0, DEPTH, level, jnp.int32(0))

    @pl.when(s == 0)
    def _():
        ifix_v[...] = jnp.zeros((16,), jnp.int32)
        pltpu.async_copy(tree_hbm.at[ifix_v], fix_v, sem).wait()
        t0 = fix_v[...]
        ifix_v[...] = jnp.ones((16,), jnp.int32)
        pltpu.async_copy(tree_hbm.at[ifix_v], fix_v, sem).wait()
        t1 = fix_v[...]
        r = t0
        for _ in range(DEPTH + 1):
            r = r + t1
        fix_v[...] = r
        pltpu.sync_copy(fix_v.at[pl.ds(0, 1)], sp.at[pl.ds(0, 1)])
    plsc.subcore_barrier()

    pltpu.sync_copy(sp.at[pl.ds(s * INT_CHUNK, INT_CHUNK)],
                    out_hbm.at[pl.ds(s * INT_CHUNK, INT_CHUNK)])


@jax.jit
def _run(tree, sidx, wvals, sibv):
    mesh = plsc.VectorSubcoreMesh(core_axis_name="c", subcore_axis_name="s",
                                  num_cores=1, num_subcores=NTILES)
    f = pl.kernel(
        _body,
        out_type=jax.ShapeDtypeStruct((TREE,), jnp.float32),
        mesh=mesh,
        scratch_types=[
            pltpu.VMEM_SHARED((CAPACITY,), jnp.float32),
            pltpu.VMEM((ROWS, 128), jnp.int32),
            pltpu.VMEM((ROWS, 128), jnp.float32),
            pltpu.VMEM((ROWS, 128), jnp.float32),
            pltpu.VMEM((ROWS, 128), jnp.int32),
            pltpu.VMEM((ROWS, 128), jnp.int32),
            pltpu.VMEM((ROWS, 128), jnp.int32),
            pltpu.VMEM((ROWS, 128), jnp.float32),
            pltpu.VMEM((ROWS, 128), jnp.float32),
            pltpu.VMEM((ROWS, 128), jnp.float32),
            pltpu.VMEM((16,), jnp.float32),
            pltpu.VMEM((16,), jnp.int32),
            pltpu.SemaphoreType.DMA,
            pltpu.SemaphoreType.DMA,
        ],
    )
    return f(tree, sidx, wvals, sibv)


def kernel(values_tree, indices, values):
    idx32 = indices.astype(jnp.int32)
    order = jnp.argsort(idx32, stable=True)
    sidx = idx32[order] + LEFTMOST
    svals = values[order].astype(jnp.float32)
    run_last = jnp.searchsorted(sidx, sidx, side="right") - 1
    wvals = svals[run_last]
    sib = sidx ^ 1
    pos = jnp.searchsorted(sidx, sib, side="right") - 1
    pos = jnp.clip(pos, 0, BATCH - 1)
    sib_updated = sidx[pos] == sib
    sibv = jnp.where(sib_updated, wvals[pos], jnp.float32(jnp.nan))
    sidx = sidx.reshape(NTILES, ROWS, 128)
    wvals = wvals.reshape(NTILES, ROWS, 128)
    sibv = sibv.reshape(NTILES, ROWS, 128)
    return _run(values_tree.astype(jnp.float32), sidx, wvals, sibv)


# speedup vs baseline: 112.3684x; 3.6581x over previous
import jax
import jax.numpy as jnp
from jax import lax
from jax.experimental import pallas as pl
from jax.experimental.pallas import tpu as pltpu
from jax.experimental.pallas import tpu_sc as plsc

CAPACITY = 1048576
LEFTMOST = 1048576
TREE = 2 * CAPACITY
DEPTH = 19
BATCH = 16384
NTILES = 16
PER_TILE = BATCH // NTILES
ROWS = PER_TILE // 128
SUB = 128 // 16
LEAF_CHUNK = CAPACITY // NTILES
INT_CHUNK = CAPACITY // NTILES


def _body(tree_hbm, sidx_hbm, nxt_hbm, sval_hbm, out_hbm,
          sp, idx_v, nxt_v, val_v, sidx2_v, pidx_v, lidx_v, ridx_v,
          left_v, right_v, sum_v, fix_v, ifix_v, sem, sem_cp):
    s = lax.axis_index("s").astype(jnp.int32)
    J = [jnp.int32(j) for j in range(ROWS)]
    one = jnp.ones((16,), jnp.int32)
    zero_i = jnp.zeros((16,), jnp.int32)
    zero_f = jnp.zeros((16,), jnp.float32)

    cp = pltpu.async_copy(
        tree_hbm.at[pl.ds(CAPACITY + s * LEAF_CHUNK, LEAF_CHUNK)],
        out_hbm.at[pl.ds(CAPACITY + s * LEAF_CHUNK, LEAF_CHUNK)], sem_cp)
    st = pltpu.async_copy(
        tree_hbm.at[pl.ds(s * INT_CHUNK, INT_CHUNK)],
        sp.at[pl.ds(s * INT_CHUNK, INT_CHUNK)], sem_cp)
    pltpu.sync_copy(sidx_hbm.at[s], idx_v)
    pltpu.sync_copy(nxt_hbm.at[s], nxt_v)
    pltpu.sync_copy(sval_hbm.at[s], val_v)
    for jj in J:
        for i in range(SUB):
            sl = pl.ds(i * 16, 16)
            v = idx_v[jj, sl]
            win = v != nxt_v[jj, sl]
            sidx2_v[jj, sl] = jnp.where(win, v, zero_i)
    cp.wait()
    st.wait()
    plsc.subcore_barrier()

    leaf = [pltpu.async_copy(val_v.at[jj], out_hbm.at[sidx2_v.at[jj]], sem_cp)
            for jj in J]

    for jj in J:
        for i in range(SUB):
            sl = pl.ds(i * 16, 16)
            v = idx_v[jj, sl]
            lidx_v[jj, sl] = lax.bitwise_and(v, ~one)
            ridx_v[jj, sl] = lax.bitwise_or(v, one)
            pidx_v[jj, sl] = lax.shift_right_logical(v, one)
    g1 = []
    for jj in J:
        g1.append(pltpu.async_copy(tree_hbm.at[lidx_v.at[jj]], left_v.at[jj], sem))
        g1.append(pltpu.async_copy(tree_hbm.at[ridx_v.at[jj]], right_v.at[jj], sem))
    for h in g1:
        h.wait()
    for jj in J:
        for i in range(SUB):
            sl = pl.ds(i * 16, 16)
            sum_v[jj, sl] = left_v[jj, sl] + right_v[jj, sl]
    s1 = [pltpu.async_copy(sum_v.at[jj], sp.at[pidx_v.at[jj]], sem)
          for jj in J]
    for h in s1:
        h.wait()
    plsc.subcore_barrier()

    for jj in J:
        for i in range(SUB):
            sl = pl.ds(i * 16, 16)
            v = idx_v[jj, sl]
            odd = lax.bitwise_and(v, one) == one
            orig = jnp.where(odd, right_v[jj, sl], left_v[jj, sl])
            win = v != nxt_v[jj, sl]
            sum_v[jj, sl] = jnp.where(win, val_v[jj, sl] - orig, zero_f)
    s2 = [pltpu.async_copy(sum_v.at[jj], sp.at[pidx_v.at[jj]], sem, add=True)
          for jj in J]
    for h in s2:
        h.wait()
    for h in leaf:
        h.wait()
    plsc.subcore_barrier()

    def level(k, carry):
        for jj in J:
            for i in range(SUB):
                sl = pl.ds(i * 16, 16)
                v = pidx_v[jj, sl]
                lidx_v[jj, sl] = lax.bitwise_and(v, ~one)
                ridx_v[jj, sl] = lax.bitwise_or(v, one)
                pidx_v[jj, sl] = lax.shift_right_logical(v, one)
        gathers = []
        for jj in J:
            gathers.append(
                pltpu.async_copy(sp.at[lidx_v.at[jj]], left_v.at[jj], sem))
            gathers.append(
                pltpu.async_copy(sp.at[ridx_v.at[jj]], right_v.at[jj], sem))
        for h in gathers:
            h.wait()
        for jj in J:
            for i in range(SUB):
                sl = pl.ds(i * 16, 16)
                sum_v[jj, sl] = left_v[jj, sl] + right_v[jj, sl]
        scatters = [pltpu.async_copy(sum_v.at[jj], sp.at[pidx_v.at[jj]], sem)
                    for jj in J]
        for h in scatters:
            h.wait()
        plsc.subcore_barrier()
        return carry

    lax.fori_loop(0, DEPTH, level, jnp.int32(0))

    @pl.when(s == 0)
    def _():
        ifix_v[...] = zero_i
        pltpu.async_copy(tree_hbm.at[ifix_v], fix_v, sem).wait()
        t0 = fix_v[...]
        ifix_v[...] = one
        pltpu.async_copy(tree_hbm.at[ifix_v], fix_v, sem).wait()
        t1 = fix_v[...]
        r = t0
        for _ in range(DEPTH + 1):
            r = r + t1
        fix_v[...] = r
        pltpu.sync_copy(fix_v.at[pl.ds(0, 1)], sp.at[pl.ds(0, 1)])
    plsc.subcore_barrier()

    pltpu.sync_copy(sp.at[pl.ds(s * INT_CHUNK, INT_CHUNK)],
                    out_hbm.at[pl.ds(s * INT_CHUNK, INT_CHUNK)])


@jax.jit
def _run(tree, sidx, nxt, svals):
    mesh = plsc.VectorSubcoreMesh(core_axis_name="c", subcore_axis_name="s",
                                  num_cores=1, num_subcores=NTILES)
    f = pl.kernel(
        _body,
        out_type=jax.ShapeDtypeStruct((TREE,), jnp.float32),
        mesh=mesh,
        scratch_types=[
            pltpu.VMEM_SHARED((CAPACITY,), jnp.float32),
            pltpu.VMEM((ROWS, 128), jnp.int32),
            pltpu.VMEM((ROWS, 128), jnp.int32),
            pltpu.VMEM((ROWS, 128), jnp.float32),
            pltpu.VMEM((ROWS, 128), jnp.int32),
            pltpu.VMEM((ROWS, 128), jnp.int32),
            pltpu.VMEM((ROWS, 128), jnp.int32),
            pltpu.VMEM((ROWS, 128), jnp.int32),
            pltpu.VMEM((ROWS, 128), jnp.float32),
            pltpu.VMEM((ROWS, 128), jnp.float32),
            pltpu.VMEM((ROWS, 128), jnp.float32),
            pltpu.VMEM((16,), jnp.float32),
            pltpu.VMEM((16,), jnp.int32),
            pltpu.SemaphoreType.DMA,
            pltpu.SemaphoreType.DMA,
        ],
    )
    return f(tree, sidx, nxt, svals)


def kernel(values_tree, indices, values):
    leaf = indices.astype(jnp.int32) + LEFTMOST
    sidx, svals = lax.sort((leaf, values.astype(jnp.float32)),
                           num_keys=1, is_stable=True)
    nxt = jnp.concatenate([sidx[1:], jnp.full((1,), -1, jnp.int32)])
    sidx = sidx.reshape(NTILES, ROWS, 128)
    nxt = nxt.reshape(NTILES, ROWS, 128)
    svals = svals.reshape(NTILES, ROWS, 128)
    return _run(values_tree.astype(jnp.float32), sidx, nxt, svals)


# speedup vs baseline: 147.4398x; 1.3121x over previous
import jax
import jax.numpy as jnp
from jax import lax
from jax.experimental import pallas as pl
from jax.experimental.pallas import tpu as pltpu
from jax.experimental.pallas import tpu_sc as plsc

CAPACITY = 1048576
LEFTMOST = 1048576
TREE = 2 * CAPACITY
DEPTH = 19
BATCH = 16384
NTILES = 16
PER_TILE = BATCH // NTILES
ROWS = PER_TILE // 128
SUB = 128 // 16
LEAF_CHUNK = CAPACITY // NTILES
INT_CHUNK = CAPACITY // NTILES


def _body(tree_hbm, sidx_hbm, nxt_hbm, sval_hbm, out_hbm,
          sp, idx_v, nxt_v, val_v, sidx2_v, pidx_v, lidx_v, ridx_v,
          left_v, right_v, sum_v, fix_v, ifix_v, sem, sem_cp):
    s = lax.axis_index("s").astype(jnp.int32)
    J = [jnp.int32(j) for j in range(ROWS)]
    one = jnp.ones((16,), jnp.int32)
    zero_i = jnp.zeros((16,), jnp.int32)
    zero_f = jnp.zeros((16,), jnp.float32)

    cp = pltpu.async_copy(
        tree_hbm.at[pl.ds(CAPACITY + s * LEAF_CHUNK, LEAF_CHUNK)],
        out_hbm.at[pl.ds(CAPACITY + s * LEAF_CHUNK, LEAF_CHUNK)], sem_cp)
    st = pltpu.async_copy(
        tree_hbm.at[pl.ds(s * INT_CHUNK, INT_CHUNK)],
        sp.at[pl.ds(s * INT_CHUNK, INT_CHUNK)], sem_cp)
    pltpu.sync_copy(sidx_hbm.at[s], idx_v)
    pltpu.sync_copy(nxt_hbm.at[s], nxt_v)
    pltpu.sync_copy(sval_hbm.at[s], val_v)
    for jj in J:
        for i in range(SUB):
            sl = pl.ds(i * 16, 16)
            v = idx_v[jj, sl]
            win = v != nxt_v[jj, sl]
            sidx2_v[jj, sl] = jnp.where(win, v, zero_i)
    cp.wait()
    st.wait()
    plsc.subcore_barrier()

    leaf = [pltpu.async_copy(val_v.at[jj], out_hbm.at[sidx2_v.at[jj]], sem_cp)
            for jj in J]

    for jj in J:
        for i in range(SUB):
            sl = pl.ds(i * 16, 16)
            v = idx_v[jj, sl]
            lidx_v[jj, sl] = lax.bitwise_and(v, ~one)
            ridx_v[jj, sl] = lax.bitwise_or(v, one)
            pidx_v[jj, sl] = lax.shift_right_logical(v, one)
    g1 = []
    for jj in J:
        g1.append(pltpu.async_copy(tree_hbm.at[lidx_v.at[jj]], left_v.at[jj], sem))
        g1.append(pltpu.async_copy(tree_hbm.at[ridx_v.at[jj]], right_v.at[jj], sem))
    for h in g1:
        h.wait()
    for jj in J:
        for i in range(SUB):
            sl = pl.ds(i * 16, 16)
            sum_v[jj, sl] = left_v[jj, sl] + right_v[jj, sl]
    s1 = [pltpu.async_copy(sum_v.at[jj], sp.at[pidx_v.at[jj]], sem)
          for jj in J]
    for h in s1:
        h.wait()
    plsc.subcore_barrier()

    for jj in J:
        for i in range(SUB):
            sl = pl.ds(i * 16, 16)
            v = idx_v[jj, sl]
            odd = lax.bitwise_and(v, one) == one
            orig = jnp.where(odd, right_v[jj, sl], left_v[jj, sl])
            win = v != nxt_v[jj, sl]
            sum_v[jj, sl] = jnp.where(win, val_v[jj, sl] - orig, zero_f)
    s2 = [pltpu.async_copy(sum_v.at[jj], sp.at[pidx_v.at[jj]], sem, add=True)
          for jj in J]
    for h in s2:
        h.wait()
    for h in leaf:
        h.wait()
    plsc.subcore_barrier()

    last = jnp.int32(15)

    def level(k, carry):
        for jj in J:
            for i in range(SUB):
                sl = pl.ds(i * 16, 16)
                v = pidx_v[jj, sl]
                lidx_v[jj, sl] = lax.bitwise_and(v, ~one)
                ridx_v[jj, sl] = lax.bitwise_or(v, one)
                pidx_v[jj, sl] = lax.shift_right_logical(v, one)
        tails = [pidx_v[jj, pl.ds(112, 16)][15] for jj in J]
        conds = [None] + [tails[r] != tails[r - 1] for r in range(1, ROWS)]

        def rowwise(fn):
            for r, jj in enumerate(J):
                if conds[r] is None:
                    fn(jj)
                else:
                    def _do(jj=jj):
                        fn(jj)
                        return None
                    pl.when(conds[r])(_do)

        def fire_gathers(jj):
            pltpu.async_copy(sp.at[lidx_v.at[jj]], left_v.at[jj], sem)
            pltpu.async_copy(sp.at[ridx_v.at[jj]], right_v.at[jj], sem)

        def wait_gathers(jj):
            pltpu.make_async_copy(tree_hbm.at[pl.ds(0, 128)],
                                  left_v.at[jj], sem).wait()
            pltpu.make_async_copy(tree_hbm.at[pl.ds(0, 128)],
                                  right_v.at[jj], sem).wait()

        def fire_scatter(jj):
            pltpu.async_copy(sum_v.at[jj], sp.at[pidx_v.at[jj]], sem)

        def wait_scatter(jj):
            pltpu.make_async_copy(tree_hbm.at[pl.ds(0, 128)],
                                  sum_v.at[jj], sem).wait()

        rowwise(fire_gathers)
        rowwise(wait_gathers)
        for jj in J:
            for i in range(SUB):
                sl = pl.ds(i * 16, 16)
                sum_v[jj, sl] = left_v[jj, sl] + right_v[jj, sl]
        rowwise(fire_scatter)
        rowwise(wait_scatter)
        plsc.subcore_barrier()
        return carry

    lax.fori_loop(0, DEPTH, level, jnp.int32(0))

    @pl.when(s == 0)
    def _():
        ifix_v[...] = zero_i
        pltpu.async_copy(tree_hbm.at[ifix_v], fix_v, sem).wait()
        t0 = fix_v[...]
        ifix_v[...] = one
        pltpu.async_copy(tree_hbm.at[ifix_v], fix_v, sem).wait()
        t1 = fix_v[...]
        r = t0
        for _ in range(DEPTH + 1):
            r = r + t1
        fix_v[...] = r
        pltpu.sync_copy(fix_v.at[pl.ds(0, 1)], sp.at[pl.ds(0, 1)])
    plsc.subcore_barrier()

    pltpu.sync_copy(sp.at[pl.ds(s * INT_CHUNK, INT_CHUNK)],
                    out_hbm.at[pl.ds(s * INT_CHUNK, INT_CHUNK)])


@jax.jit
def _run(tree, sidx, nxt, svals):
    mesh = plsc.VectorSubcoreMesh(core_axis_name="c", subcore_axis_name="s",
                                  num_cores=1, num_subcores=NTILES)
    f = pl.kernel(
        _body,
        out_type=jax.ShapeDtypeStruct((TREE,), jnp.float32),
        mesh=mesh,
        scratch_types=[
            pltpu.VMEM_SHARED((CAPACITY,), jnp.float32),
            pltpu.VMEM((ROWS, 128), jnp.int32),
            pltpu.VMEM((ROWS, 128), jnp.int32),
            pltpu.VMEM((ROWS, 128), jnp.float32),
            pltpu.VMEM((ROWS, 128), jnp.int32),
            pltpu.VMEM((ROWS, 128), jnp.int32),
            pltpu.VMEM((ROWS, 128), jnp.int32),
            pltpu.VMEM((ROWS, 128), jnp.int32),
            pltpu.VMEM((ROWS, 128), jnp.float32),
            pltpu.VMEM((ROWS, 128), jnp.float32),
            pltpu.VMEM((ROWS, 128), jnp.float32),
            pltpu.VMEM((16,), jnp.float32),
            pltpu.VMEM((16,), jnp.int32),
            pltpu.SemaphoreType.DMA,
            pltpu.SemaphoreType.DMA,
        ],
    )
    return f(tree, sidx, nxt, svals)


def kernel(values_tree, indices, values):
    leaf = indices.astype(jnp.int32) + LEFTMOST
    sidx, svals = lax.sort((leaf, values.astype(jnp.float32)),
                           num_keys=1, is_stable=True)
    nxt = jnp.concatenate([sidx[1:], jnp.full((1,), -1, jnp.int32)])
    sidx = sidx.reshape(NTILES, ROWS, 128)
    nxt = nxt.reshape(NTILES, ROWS, 128)
    svals = svals.reshape(NTILES, ROWS, 128)
    return _run(values_tree.astype(jnp.float32), sidx, nxt, svals)
